# Initial kernel scaffold; baseline (speedup 1.0000x reference)
import jax
import jax.numpy as jnp
from jax.experimental import pallas as pl


def kernel(categorical_inputs, tables):
    raise NotImplementedError("write your pallas kernel here")


# speedup vs baseline: 5.3659x; 5.3659x over previous
import functools

import jax
import jax.numpy as jnp
from jax import lax
from jax.experimental import pallas as pl
from jax.experimental.pallas import tpu as pltpu, tpu_sc as plsc

_NC = 2
_NS = 16
_NW = _NC * _NS
_PIECE = 4096
_H0 = 49920


def _make_sc_kernel(num_fields, batch, vocab, dim):
    assert dim == _NW and batch % _PIECE == 0
    npiece = batch // _PIECE
    v_main = vocab // 128 * 128
    h1_len = v_main - _H0
    h1_alloc = h1_len + 128

    mesh = plsc.VectorSubcoreMesh(core_axis_name="c", subcore_axis_name="s")

    @functools.partial(
        pl.kernel,
        mesh=mesh,
        out_type=jax.ShapeDtypeStruct((dim, batch), jnp.float32),
        compiler_params=pltpu.CompilerParams(
            use_tc_tiling_on_sc=True, needs_layout_passes=False
        ),
        scratch_types=dict(
            slab0_v=pltpu.VMEM((_H0,), jnp.float32),
            slab1_v=pltpu.VMEM((h1_alloc,), jnp.float32),
            idx_vs=[pltpu.VMEM((_PIECE,), jnp.int32) for _ in range(2)],
            acc_v=pltpu.VMEM((batch,), jnp.float32),
            slab_sems=[pltpu.SemaphoreType.DMA for _ in range(2)],
            idx_sems=[pltpu.SemaphoreType.DMA for _ in range(2)],
        ),
    )
    def sc_kernel(
        idx_hbm,
        tabt_hbm,
        tail_hbm,
        out_hbm,
        *,
        slab0_v,
        slab1_v,
        idx_vs,
        acc_v,
        slab_sems,
        idx_sems,
    ):
        e = lax.axis_index("s") * _NC + lax.axis_index("c")
        slabs = (slab0_v, slab1_v)

        def h0_copy(f):
            return pltpu.make_async_copy(
                tabt_hbm.at[f, e, pl.ds(0, _H0)],
                slab0_v,
                slab_sems[0],
            )

        def h1_copy(f):
            return pltpu.make_async_copy(
                tabt_hbm.at[f, e, pl.ds(_H0, h1_len)],
                slab1_v.at[pl.ds(0, h1_len)],
                slab_sems[1],
            )

        def tail_copy(f):
            return pltpu.make_async_copy(
                tail_hbm.at[f, e],
                slab1_v.at[pl.ds(h1_len, 128)],
                slab_sems[1],
            )

        def idx_copy(f, p, buf):
            return pltpu.make_async_copy(
                idx_hbm.at[f, pl.ds(p * _PIECE, _PIECE)],
                idx_vs[buf],
                idx_sems[buf],
            )

        h0_copy(0).start()
        h1_copy(0).start()
        tail_copy(0).start()
        idx_copy(0, 0, 0).start()

        zero = jnp.zeros((16,), jnp.float32)

        @pl.loop(0, batch // 16)
        def _(r):
            acc_v[pl.ds(r * 16, 16)] = zero

        @pl.loop(0, num_fields)
        def _(f):
            for h in range(2):
                if h == 0:
                    h0_copy(f).wait()
                else:
                    h1_copy(f).wait()
                    tail_copy(f).wait()
                lo = jnp.int32(h * _H0)

                for p in range(npiece):
                    pb = p % 2
                    nb = (p + 1) % 2
                    idx_copy(f, p, pb).wait()
                    if p < npiece - 1:
                        idx_copy(f, p + 1, nb).start()
                    elif h == 0:
                        idx_copy(f, 0, nb).start()
                    else:

                        @pl.when(f < num_fields - 1)
                        def _():
                            idx_copy(f + 1, 0, nb).start()

                    @plsc.parallel_loop(0, _PIECE // 16, unroll=8)
                    def _(i):
                        iv = idx_vs[pb][pl.ds(i * 16, 16)]
                        if h == 0:
                            mask = iv < jnp.int32(_H0)
                        else:
                            mask = iv >= jnp.int32(_H0)
                        g = plsc.load_gather(slabs[h], [iv - lo], mask=mask)
                        g = jnp.where(mask, g, jnp.float32(0.0))
                        plsc.addupdate(acc_v.at[pl.ds(p * _PIECE + i * 16, 16)], g)

                @pl.when(f < num_fields - 1)
                def _():
                    if h == 0:
                        h0_copy(f + 1).start()
                    else:
                        h1_copy(f + 1).start()
                        tail_copy(f + 1).start()

        pltpu.sync_copy(acc_v, out_hbm.at[e])

    return sc_kernel


def kernel(categorical_inputs, tables):
    num_fields, batch = categorical_inputs.shape
    _, vocab, dim = tables.shape
    v_main = vocab // 128 * 128

    tables_t = jnp.transpose(tables, (0, 2, 1))
    tail = jnp.pad(tables_t[:, :, v_main:], ((0, 0), (0, 0), (0, 128 - (vocab - v_main))))

    sc = _make_sc_kernel(num_fields, batch, vocab, dim)
    out_t = sc(categorical_inputs, tables_t, tail)
    return out_t.T


# speedup vs baseline: 6.0462x; 1.1268x over previous
import functools

import jax
import jax.numpy as jnp
from jax import lax
from jax.experimental import pallas as pl
from jax.experimental.pallas import tpu as pltpu, tpu_sc as plsc

_NC = 2
_NS = 16
_NW = _NC * _NS
_PIECE = 4096
_H0 = 49920


def _make_sc_kernel(num_fields, batch, vocab, dim):
    assert dim == _NW and batch % _PIECE == 0
    npiece = batch // _PIECE
    v_main = vocab // 128 * 128
    h1_len = v_main - _H0
    h1_alloc = h1_len + 128

    mesh = plsc.VectorSubcoreMesh(core_axis_name="c", subcore_axis_name="s")

    @functools.partial(
        pl.kernel,
        mesh=mesh,
        out_type=jax.ShapeDtypeStruct((dim, batch), jnp.float32),
        compiler_params=pltpu.CompilerParams(
            use_tc_tiling_on_sc=True, needs_layout_passes=False
        ),
        scratch_types=dict(
            slab_v=pltpu.VMEM((_H0 + h1_alloc,), jnp.float32),
            idx_vs=[pltpu.VMEM((_PIECE,), jnp.int32) for _ in range(2)],
            acc_v=pltpu.VMEM((batch,), jnp.float32),
            slab_sems=[pltpu.SemaphoreType.DMA for _ in range(2)],
            idx_sems=[pltpu.SemaphoreType.DMA for _ in range(2)],
        ),
    )
    def sc_kernel(
        idx_hbm,
        tabt_hbm,
        tail_hbm,
        out_hbm,
        *,
        slab_v,
        idx_vs,
        acc_v,
        slab_sems,
        idx_sems,
    ):
        e = lax.axis_index("s") * _NC + lax.axis_index("c")

        def h0_copy(f):
            return pltpu.make_async_copy(
                tabt_hbm.at[f, e, pl.ds(0, _H0)],
                slab_v.at[pl.ds(0, _H0)],
                slab_sems[0],
            )

        def h1_copy(f):
            return pltpu.make_async_copy(
                tabt_hbm.at[f, e, pl.ds(_H0, h1_len)],
                slab_v.at[pl.ds(_H0, h1_len)],
                slab_sems[1],
            )

        def tail_copy(f):
            return pltpu.make_async_copy(
                tail_hbm.at[f, e],
                slab_v.at[pl.ds(_H0 + h1_len, 128)],
                slab_sems[1],
            )

        def idx_copy(f, p, buf):
            return pltpu.make_async_copy(
                idx_hbm.at[pl.ds(f * batch + p * _PIECE, _PIECE)],
                idx_vs[buf],
                idx_sems[buf],
            )

        h0_copy(0).start()
        h1_copy(0).start()
        tail_copy(0).start()
        idx_copy(0, 0, 0).start()

        zero = jnp.zeros((16,), jnp.float32)

        @pl.loop(0, batch // 16)
        def _(r):
            acc_v[pl.ds(r * 16, 16)] = zero

        @pl.loop(0, num_fields)
        def _(f):
            h0_copy(f).wait()
            h1_copy(f).wait()
            tail_copy(f).wait()

            for p in range(npiece):
                pb = p % 2
                nb = (p + 1) % 2
                idx_copy(f, p, pb).wait()
                if p < npiece - 1:
                    idx_copy(f, p + 1, nb).start()
                else:

                    @pl.when(f < num_fields - 1)
                    def _():
                        idx_copy(f + 1, 0, nb).start()

                @plsc.parallel_loop(0, _PIECE // 16, unroll=8)
                def _(i):
                    iv = idx_vs[pb][pl.ds(i * 16, 16)]
                    g = plsc.load_gather(slab_v, [iv])
                    plsc.addupdate(acc_v.at[pl.ds(p * _PIECE + i * 16, 16)], g)

            @pl.when(f < num_fields - 1)
            def _():
                h0_copy(f + 1).start()
                h1_copy(f + 1).start()
                tail_copy(f + 1).start()

        pltpu.sync_copy(acc_v, out_hbm.at[e])

    return sc_kernel


def kernel(categorical_inputs, tables):
    num_fields, batch = categorical_inputs.shape
    _, vocab, dim = tables.shape
    v_main = vocab // 128 * 128

    tables_t = jnp.transpose(tables, (0, 2, 1))
    tail = jnp.pad(tables_t[:, :, v_main:], ((0, 0), (0, 0), (0, 128 - (vocab - v_main))))

    sc = _make_sc_kernel(num_fields, batch, vocab, dim)
    out_t = sc(categorical_inputs.reshape(-1), tables_t, tail)
    return out_t.T
